# baseline (speedup 1.0000x reference)
import functools

import jax
import jax.numpy as jnp
from jax import lax
from jax.experimental import pallas as pl
from jax.experimental.pallas import tpu as pltpu
from jax.experimental.pallas import tpu_sc as plsc

N = 50000
E = 800000
IN_FEAT = 128
NHID = 64
H = NHID // 2
NREL = 3
CHUNK = 128
ROWS = E // CHUNK
NTILES = 16
ROWS_PER_TILE = ROWS // NTILES
ROWS_EXTRA = ROWS % NTILES
NODE_ROWS_PER_TILE = N // NTILES
G = 10
NGRP = ROWS_PER_TILE // G

BN = 2000


def _stage0_body(x_ref, embW_ref, embb_ref, rW_ref, rb_ref, w0_ref, w1_ref,
                 w2_ref, rlo_ref, rhi_ref, mlo_ref, mhi_ref):
    h = jnp.dot(x_ref[...], embW_ref[...],
                preferred_element_type=jnp.float32) + embb_ref[...]
    r = (jnp.dot(h, rW_ref[...], preferred_element_type=jnp.float32)
         + rb_ref[...]).astype(jnp.bfloat16)
    rlo_ref[...] = r[:, :H]
    rhi_ref[...] = r[:, H:]
    for i, w_ref in enumerate((w0_ref, w1_ref, w2_ref)):
        m = jnp.dot(h, w_ref[...],
                    preferred_element_type=jnp.float32).astype(jnp.bfloat16)
        mlo_ref[i] = m[:, :H]
        mhi_ref[i] = m[:, H:]


def _stage1_body(olo_ref, ohi_ref, rW_ref, rb_ref, w0_ref, w1_ref, w2_ref,
                 rlo_ref, rhi_ref, mlo_ref, mhi_ref):
    h = jnp.maximum(
        jnp.concatenate([olo_ref[...], ohi_ref[...]],
                        axis=1).astype(jnp.float32), 0.0)
    r = (jnp.dot(h, rW_ref[...], preferred_element_type=jnp.float32)
         + rb_ref[...]).astype(jnp.bfloat16)
    rlo_ref[...] = r[:, :H]
    rhi_ref[...] = r[:, H:]
    for i, w_ref in enumerate((w0_ref, w1_ref, w2_ref)):
        m = jnp.dot(h, w_ref[...],
                    preferred_element_type=jnp.float32).astype(jnp.bfloat16)
        mlo_ref[i] = m[:, :H]
        mhi_ref[i] = m[:, H:]


def _stage2_body(olo_ref, ohi_ref, w1_ref, b1_ref, w2_ref, b2_ref, o_ref):
    h = jnp.maximum(
        jnp.concatenate([olo_ref[...], ohi_ref[...]],
                        axis=1).astype(jnp.float32), 0.0)
    g = jnp.sum(h, axis=0, keepdims=True) * (1.0 / N)
    hh = jnp.maximum(
        jnp.dot(g, w1_ref[...], preferred_element_type=jnp.float32)
        + b1_ref[...], 0.0)
    o_ref[...] = (jnp.dot(hh, w2_ref[...], preferred_element_type=jnp.float32)
                  + b2_ref[...])


def _stage0(x, embW, embb, rW, rb, w0, w1, w2):
    nblk = N // BN
    return pl.pallas_call(
        _stage0_body,
        grid=(nblk,),
        in_specs=[
            pl.BlockSpec((BN, IN_FEAT), lambda b: (b, 0)),
            pl.BlockSpec((IN_FEAT, NHID), lambda b: (0, 0)),
            pl.BlockSpec((1, NHID), lambda b: (0, 0)),
            pl.BlockSpec((NHID, NHID), lambda b: (0, 0)),
            pl.BlockSpec((1, NHID), lambda b: (0, 0)),
            pl.BlockSpec((NHID, NHID), lambda b: (0, 0)),
            pl.BlockSpec((NHID, NHID), lambda b: (0, 0)),
            pl.BlockSpec((NHID, NHID), lambda b: (0, 0)),
        ],
        out_specs=[
            pl.BlockSpec((BN, H), lambda b: (b, 0)),
            pl.BlockSpec((BN, H), lambda b: (b, 0)),
            pl.BlockSpec((NREL, BN, H), lambda b: (0, b, 0)),
            pl.BlockSpec((NREL, BN, H), lambda b: (0, b, 0)),
        ],
        out_shape=[
            jax.ShapeDtypeStruct((N, H), jnp.bfloat16),
            jax.ShapeDtypeStruct((N, H), jnp.bfloat16),
            jax.ShapeDtypeStruct((NREL, N, H), jnp.bfloat16),
            jax.ShapeDtypeStruct((NREL, N, H), jnp.bfloat16),
        ],
    )(x, embW, embb, rW, rb, w0, w1, w2)


def _stage1(olo, ohi, rW, rb, w0, w1, w2):
    nblk = N // BN
    return pl.pallas_call(
        _stage1_body,
        grid=(nblk,),
        in_specs=[
            pl.BlockSpec((BN, H), lambda b: (b, 0)),
            pl.BlockSpec((BN, H), lambda b: (b, 0)),
            pl.BlockSpec((NHID, NHID), lambda b: (0, 0)),
            pl.BlockSpec((1, NHID), lambda b: (0, 0)),
            pl.BlockSpec((NHID, NHID), lambda b: (0, 0)),
            pl.BlockSpec((NHID, NHID), lambda b: (0, 0)),
            pl.BlockSpec((NHID, NHID), lambda b: (0, 0)),
        ],
        out_specs=[
            pl.BlockSpec((BN, H), lambda b: (b, 0)),
            pl.BlockSpec((BN, H), lambda b: (b, 0)),
            pl.BlockSpec((NREL, BN, H), lambda b: (0, b, 0)),
            pl.BlockSpec((NREL, BN, H), lambda b: (0, b, 0)),
        ],
        out_shape=[
            jax.ShapeDtypeStruct((N, H), jnp.bfloat16),
            jax.ShapeDtypeStruct((N, H), jnp.bfloat16),
            jax.ShapeDtypeStruct((NREL, N, H), jnp.bfloat16),
            jax.ShapeDtypeStruct((NREL, N, H), jnp.bfloat16),
        ],
    )(olo, ohi, rW, rb, w0, w1, w2)


def _stage2(olo, ohi, w1, b1, w2, b2):
    return pl.pallas_call(
        _stage2_body,
        out_shape=jax.ShapeDtypeStruct((1, 1), jnp.float32),
    )(olo, ohi, w1, b1, w2, b2)


def _sc_scatter(src0, dst0, src1, dst1, src2, dst2, r_lo, r_hi, m_lo, m_hi):
    mesh = plsc.VectorSubcoreMesh(core_axis_name="c", subcore_axis_name="s")

    @functools.partial(
        pl.kernel,
        mesh=mesh,
        out_type=[jax.ShapeDtypeStruct((N, H), jnp.bfloat16),
                  jax.ShapeDtypeStruct((N, H), jnp.bfloat16)],
        scratch_types=[
            pltpu.VMEM((2, G, CHUNK), jnp.int32),
            pltpu.VMEM((2, G, CHUNK), jnp.int32),
            pltpu.VMEM((2, G, CHUNK, H), jnp.bfloat16),
            pltpu.VMEM_SHARED((N, H), jnp.bfloat16),
        ],
        compiler_params=pltpu.CompilerParams(use_tc_tiling_on_sc=False),
    )
    def scatter_kernel(s0, d0, s1, d1, s2, d2, rlo, rhi, mlo, mhi,
                       out_lo, out_hi, sv, dv, rows, acc):
        c = lax.axis_index("c")
        s = lax.axis_index("s")
        node_lo = s * NODE_ROWS_PER_TILE

        def half(r_ref, m_ref, out_ref, sem_i, sem_g, sem_s):
            pltpu.sync_copy(r_ref.at[pl.ds(node_lo, NODE_ROWS_PER_TILE)],
                            acc.at[pl.ds(node_lo, NODE_ROWS_PER_TILE)])
            plsc.subcore_barrier()
            row0 = s * ROWS_PER_TILE
            for i, (sr, dr) in enumerate(((s0, d0), (s1, d1), (s2, d2))):
                table = m_ref.at[i]

                def body(g, _, sr=sr, dr=dr, table=table):
                    p = lax.rem(g, 2)
                    base = row0 + g * G
                    i1 = pltpu.async_copy(sr.at[pl.ds(base, G)], sv.at[p],
                                          sem_i)
                    i2 = pltpu.async_copy(dr.at[pl.ds(base, G)], dv.at[p],
                                          sem_i)
                    i1.wait()
                    i2.wait()
                    gathers = [
                        pltpu.async_copy(table.at[sv.at[p, k]],
                                         rows.at[p, k], sem_g)
                        for k in range(G)]
                    for h_ in gathers:
                        h_.wait()
                    scats = [
                        pltpu.async_copy(rows.at[p, k], acc.at[dv.at[p, k]],
                                         sem_s, add=True)
                        for k in range(G)]
                    for h_ in scats:
                        h_.wait()
                    return 0

                lax.fori_loop(0, NGRP, body, 0)

                @pl.when(s < ROWS_EXTRA)
                def _(sr=sr, dr=dr, table=table):
                    er = NTILES * ROWS_PER_TILE + s
                    pltpu.sync_copy(sr.at[er], sv.at[0, 0])
                    pltpu.sync_copy(dr.at[er], dv.at[0, 0])
                    pltpu.sync_copy(table.at[sv.at[0, 0]], rows.at[0, 0])
                    pltpu.sync_copy(rows.at[0, 0], acc.at[dv.at[0, 0]],
                                    add=True)
            plsc.subcore_barrier()
            pltpu.sync_copy(acc.at[pl.ds(node_lo, NODE_ROWS_PER_TILE)],
                            out_ref.at[pl.ds(node_lo, NODE_ROWS_PER_TILE)])

        @pl.when(c == 0)
        def _():
            pl.run_scoped(
                lambda sem_i, sem_g, sem_s: half(rlo, mlo, out_lo,
                                                 sem_i, sem_g, sem_s),
                pltpu.SemaphoreType.DMA, pltpu.SemaphoreType.DMA,
                pltpu.SemaphoreType.DMA)

        @pl.when(c == 1)
        def _():
            pl.run_scoped(
                lambda sem_i, sem_g, sem_s: half(rhi, mhi, out_hi,
                                                 sem_i, sem_g, sem_s),
                pltpu.SemaphoreType.DMA, pltpu.SemaphoreType.DMA,
                pltpu.SemaphoreType.DMA)

    return scatter_kernel(src0, dst0, src1, dst1, src2, dst2,
                          r_lo, r_hi, m_lo, m_hi)


def kernel(x, edge_index_0, edge_index_1, edge_index_2, batch, emb_W, emb_b,
           root_W_0, root_b_0, conv_W_0_0, conv_W_0_1, conv_W_0_2,
           root_W_1, root_b_1, conv_W_1_0, conv_W_1_1, conv_W_1_2,
           mlp_W1, mlp_b1, mlp_W2, mlp_b2):
    del batch
    edges = []
    for ei in (edge_index_0, edge_index_1, edge_index_2):
        edges.append(ei[0].reshape(ROWS, CHUNK))
        edges.append(ei[1].reshape(ROWS, CHUNK))

    r_lo, r_hi, m_lo, m_hi = _stage0(
        x, emb_W, emb_b.reshape(1, NHID), root_W_0, root_b_0.reshape(1, NHID),
        conv_W_0_0, conv_W_0_1, conv_W_0_2)
    out_lo, out_hi = _sc_scatter(*edges, r_lo, r_hi, m_lo, m_hi)

    r_lo, r_hi, m_lo, m_hi = _stage1(
        out_lo, out_hi, root_W_1, root_b_1.reshape(1, NHID),
        conv_W_1_0, conv_W_1_1, conv_W_1_2)
    out_lo, out_hi = _sc_scatter(*edges, r_lo, r_hi, m_lo, m_hi)

    o = _stage2(out_lo, out_hi, mlp_W1, mlp_b1.reshape(1, NHID),
                mlp_W2, mlp_b2.reshape(1, 1))
    return o.reshape(1)


# speedup vs baseline: 1.0994x; 1.0994x over previous
import functools

import jax
import jax.numpy as jnp
from jax import lax
from jax.experimental import pallas as pl
from jax.experimental.pallas import tpu as pltpu
from jax.experimental.pallas import tpu_sc as plsc

N = 50000
E = 800000
IN_FEAT = 128
NHID = 64
H = NHID // 2
NREL = 3
CHUNK = 128
ROWS = E // CHUNK
NTILES = 16
ROWS_PER_TILE = ROWS // NTILES
ROWS_EXTRA = ROWS % NTILES
NODE_ROWS_PER_TILE = N // NTILES
G = 10
NGRP = ROWS_PER_TILE // G

BN = 2000


def _stage0_body(x_ref, embW_ref, embb_ref, rW_ref, rb_ref, w0_ref, w1_ref,
                 w2_ref, rlo_ref, rhi_ref, mlo_ref, mhi_ref):
    h = jnp.dot(x_ref[...], embW_ref[...],
                preferred_element_type=jnp.float32) + embb_ref[...]
    r = (jnp.dot(h, rW_ref[...], preferred_element_type=jnp.float32)
         + rb_ref[...]).astype(jnp.bfloat16)
    rlo_ref[...] = r[:, :H]
    rhi_ref[...] = r[:, H:]
    for i, w_ref in enumerate((w0_ref, w1_ref, w2_ref)):
        m = jnp.dot(h, w_ref[...],
                    preferred_element_type=jnp.float32).astype(jnp.bfloat16)
        mlo_ref[i] = m[:, :H]
        mhi_ref[i] = m[:, H:]


def _stage1_body(olo_ref, ohi_ref, rW_ref, rb_ref, w0_ref, w1_ref, w2_ref,
                 rlo_ref, rhi_ref, mlo_ref, mhi_ref):
    h = jnp.maximum(
        jnp.concatenate([olo_ref[...], ohi_ref[...]],
                        axis=1).astype(jnp.float32), 0.0)
    r = (jnp.dot(h, rW_ref[...], preferred_element_type=jnp.float32)
         + rb_ref[...]).astype(jnp.bfloat16)
    rlo_ref[...] = r[:, :H]
    rhi_ref[...] = r[:, H:]
    for i, w_ref in enumerate((w0_ref, w1_ref, w2_ref)):
        m = jnp.dot(h, w_ref[...],
                    preferred_element_type=jnp.float32).astype(jnp.bfloat16)
        mlo_ref[i] = m[:, :H]
        mhi_ref[i] = m[:, H:]


def _stage2_body(olo_ref, ohi_ref, w1_ref, b1_ref, w2_ref, b2_ref, o_ref):
    h = jnp.maximum(
        jnp.concatenate([olo_ref[...], ohi_ref[...]],
                        axis=1).astype(jnp.float32), 0.0)
    g = jnp.sum(h, axis=0, keepdims=True) * (1.0 / N)
    hh = jnp.maximum(
        jnp.dot(g, w1_ref[...], preferred_element_type=jnp.float32)
        + b1_ref[...], 0.0)
    o_ref[...] = (jnp.dot(hh, w2_ref[...], preferred_element_type=jnp.float32)
                  + b2_ref[...])


def _stage0(x, embW, embb, rW, rb, w0, w1, w2):
    nblk = N // BN
    return pl.pallas_call(
        _stage0_body,
        grid=(nblk,),
        in_specs=[
            pl.BlockSpec((BN, IN_FEAT), lambda b: (b, 0)),
            pl.BlockSpec((IN_FEAT, NHID), lambda b: (0, 0)),
            pl.BlockSpec((1, NHID), lambda b: (0, 0)),
            pl.BlockSpec((NHID, NHID), lambda b: (0, 0)),
            pl.BlockSpec((1, NHID), lambda b: (0, 0)),
            pl.BlockSpec((NHID, NHID), lambda b: (0, 0)),
            pl.BlockSpec((NHID, NHID), lambda b: (0, 0)),
            pl.BlockSpec((NHID, NHID), lambda b: (0, 0)),
        ],
        out_specs=[
            pl.BlockSpec((BN, H), lambda b: (b, 0)),
            pl.BlockSpec((BN, H), lambda b: (b, 0)),
            pl.BlockSpec((NREL, BN, H), lambda b: (0, b, 0)),
            pl.BlockSpec((NREL, BN, H), lambda b: (0, b, 0)),
        ],
        out_shape=[
            jax.ShapeDtypeStruct((N, H), jnp.bfloat16),
            jax.ShapeDtypeStruct((N, H), jnp.bfloat16),
            jax.ShapeDtypeStruct((NREL, N, H), jnp.bfloat16),
            jax.ShapeDtypeStruct((NREL, N, H), jnp.bfloat16),
        ],
    )(x, embW, embb, rW, rb, w0, w1, w2)


def _stage1(olo, ohi, rW, rb, w0, w1, w2):
    nblk = N // BN
    return pl.pallas_call(
        _stage1_body,
        grid=(nblk,),
        in_specs=[
            pl.BlockSpec((BN, H), lambda b: (b, 0)),
            pl.BlockSpec((BN, H), lambda b: (b, 0)),
            pl.BlockSpec((NHID, NHID), lambda b: (0, 0)),
            pl.BlockSpec((1, NHID), lambda b: (0, 0)),
            pl.BlockSpec((NHID, NHID), lambda b: (0, 0)),
            pl.BlockSpec((NHID, NHID), lambda b: (0, 0)),
            pl.BlockSpec((NHID, NHID), lambda b: (0, 0)),
        ],
        out_specs=[
            pl.BlockSpec((BN, H), lambda b: (b, 0)),
            pl.BlockSpec((BN, H), lambda b: (b, 0)),
            pl.BlockSpec((NREL, BN, H), lambda b: (0, b, 0)),
            pl.BlockSpec((NREL, BN, H), lambda b: (0, b, 0)),
        ],
        out_shape=[
            jax.ShapeDtypeStruct((N, H), jnp.bfloat16),
            jax.ShapeDtypeStruct((N, H), jnp.bfloat16),
            jax.ShapeDtypeStruct((NREL, N, H), jnp.bfloat16),
            jax.ShapeDtypeStruct((NREL, N, H), jnp.bfloat16),
        ],
    )(olo, ohi, rW, rb, w0, w1, w2)


def _stage2(olo, ohi, w1, b1, w2, b2):
    return pl.pallas_call(
        _stage2_body,
        out_shape=jax.ShapeDtypeStruct((1, 1), jnp.float32),
    )(olo, ohi, w1, b1, w2, b2)


def _sc_scatter(src0, dst0, src1, dst1, src2, dst2, r_lo, r_hi, m_lo, m_hi):
    mesh = plsc.VectorSubcoreMesh(core_axis_name="c", subcore_axis_name="s")

    @functools.partial(
        pl.kernel,
        mesh=mesh,
        out_type=[jax.ShapeDtypeStruct((N, H), jnp.bfloat16),
                  jax.ShapeDtypeStruct((N, H), jnp.bfloat16)],
        scratch_types=[
            pltpu.VMEM((2, G, CHUNK), jnp.int32),
            pltpu.VMEM((2, G, CHUNK), jnp.int32),
            pltpu.VMEM((2, G, CHUNK, H), jnp.bfloat16),
            pltpu.VMEM_SHARED((N, H), jnp.bfloat16),
            pltpu.SemaphoreType.DMA,
            pltpu.SemaphoreType.DMA,
            pltpu.SemaphoreType.DMA,
        ],
        compiler_params=pltpu.CompilerParams(use_tc_tiling_on_sc=False),
    )
    def scatter_kernel(s0, d0, s1, d1, s2, d2, rlo, rhi, mlo, mhi,
                       out_lo, out_hi, sv, dv, rows, acc, sem_i, sem_g, sem_s):
        c = lax.axis_index("c")
        s = lax.axis_index("s")
        node_lo = s * NODE_ROWS_PER_TILE

        def half(r_ref, m_ref, out_ref):
            pltpu.sync_copy(r_ref.at[pl.ds(node_lo, NODE_ROWS_PER_TILE)],
                            acc.at[pl.ds(node_lo, NODE_ROWS_PER_TILE)])
            plsc.subcore_barrier()
            row0 = s * ROWS_PER_TILE
            for i, (sr, dr) in enumerate(((s0, d0), (s1, d1), (s2, d2))):
                table = m_ref.at[i]

                def issue_idx(g, p, sr=sr, dr=dr):
                    base = row0 + g * G
                    pltpu.async_copy(sr.at[pl.ds(base, G)], sv.at[p], sem_i)
                    pltpu.async_copy(dr.at[pl.ds(base, G)], dv.at[p], sem_i)

                issue_idx(0, 0)

                def body(g, _, sr=sr, dr=dr, table=table, issue_idx=issue_idx):
                    p = lax.rem(g, 2)
                    base = row0 + g * G
                    pltpu.make_async_copy(sr.at[pl.ds(base, G)], sv.at[p],
                                          sem_i).wait()
                    pltpu.make_async_copy(dr.at[pl.ds(base, G)], dv.at[p],
                                          sem_i).wait()
                    gathers = [
                        pltpu.async_copy(table.at[sv.at[p, k]],
                                         rows.at[p, k], sem_g)
                        for k in range(G)]

                    @pl.when(g < NGRP - 1)
                    def _():
                        issue_idx(g + 1, 1 - p)

                    for h_ in gathers:
                        h_.wait()
                    scats = [
                        pltpu.async_copy(rows.at[p, k], acc.at[dv.at[p, k]],
                                         sem_s, add=True)
                        for k in range(G)]
                    for h_ in scats:
                        h_.wait()
                    return 0

                lax.fori_loop(0, NGRP, body, 0)

                @pl.when(s < ROWS_EXTRA)
                def _(sr=sr, dr=dr, table=table):
                    er = NTILES * ROWS_PER_TILE + s
                    pltpu.sync_copy(sr.at[er], sv.at[0, 0])
                    pltpu.sync_copy(dr.at[er], dv.at[0, 0])
                    pltpu.sync_copy(table.at[sv.at[0, 0]], rows.at[0, 0])
                    pltpu.sync_copy(rows.at[0, 0], acc.at[dv.at[0, 0]],
                                    add=True)
            plsc.subcore_barrier()
            pltpu.sync_copy(acc.at[pl.ds(node_lo, NODE_ROWS_PER_TILE)],
                            out_ref.at[pl.ds(node_lo, NODE_ROWS_PER_TILE)])

        @pl.when(c == 0)
        def _():
            half(rlo, mlo, out_lo)

        @pl.when(c == 1)
        def _():
            half(rhi, mhi, out_hi)

    return scatter_kernel(src0, dst0, src1, dst1, src2, dst2,
                          r_lo, r_hi, m_lo, m_hi)


def kernel(x, edge_index_0, edge_index_1, edge_index_2, batch, emb_W, emb_b,
           root_W_0, root_b_0, conv_W_0_0, conv_W_0_1, conv_W_0_2,
           root_W_1, root_b_1, conv_W_1_0, conv_W_1_1, conv_W_1_2,
           mlp_W1, mlp_b1, mlp_W2, mlp_b2):
    del batch
    edges = []
    for ei in (edge_index_0, edge_index_1, edge_index_2):
        edges.append(ei[0].reshape(ROWS, CHUNK))
        edges.append(ei[1].reshape(ROWS, CHUNK))

    r_lo, r_hi, m_lo, m_hi = _stage0(
        x, emb_W, emb_b.reshape(1, NHID), root_W_0, root_b_0.reshape(1, NHID),
        conv_W_0_0, conv_W_0_1, conv_W_0_2)
    out_lo, out_hi = _sc_scatter(*edges, r_lo, r_hi, m_lo, m_hi)

    r_lo, r_hi, m_lo, m_hi = _stage1(
        out_lo, out_hi, root_W_1, root_b_1.reshape(1, NHID),
        conv_W_1_0, conv_W_1_1, conv_W_1_2)
    out_lo, out_hi = _sc_scatter(*edges, r_lo, r_hi, m_lo, m_hi)

    o = _stage2(out_lo, out_hi, mlp_W1, mlp_b1.reshape(1, NHID),
                mlp_W2, mlp_b2.reshape(1, 1))
    return o.reshape(1)


# speedup vs baseline: 1.1437x; 1.0402x over previous
import functools

import jax
import jax.numpy as jnp
from jax import lax
from jax.experimental import pallas as pl
from jax.experimental.pallas import tpu as pltpu
from jax.experimental.pallas import tpu_sc as plsc

N = 50000
E = 800000
IN_FEAT = 128
NHID = 64
H = NHID // 2
NREL = 3
CHUNK = 128
ROWS = E // CHUNK
NTILES = 16
ROWS_PER_TILE = ROWS // NTILES
ROWS_EXTRA = ROWS % NTILES
NODE_ROWS_PER_TILE = N // NTILES
G = 15
NGRP = ROWS_PER_TILE // G

BN = 2000


def _stage0_body(x_ref, embW_ref, embb_ref, rW_ref, rb_ref, w0_ref, w1_ref,
                 w2_ref, rlo_ref, rhi_ref, mlo_ref, mhi_ref):
    h = jnp.dot(x_ref[...], embW_ref[...],
                preferred_element_type=jnp.float32) + embb_ref[...]
    r = (jnp.dot(h, rW_ref[...], preferred_element_type=jnp.float32)
         + rb_ref[...]).astype(jnp.bfloat16)
    rlo_ref[...] = r[:, :H]
    rhi_ref[...] = r[:, H:]
    for i, w_ref in enumerate((w0_ref, w1_ref, w2_ref)):
        m = jnp.dot(h, w_ref[...],
                    preferred_element_type=jnp.float32).astype(jnp.bfloat16)
        mlo_ref[i] = m[:, :H]
        mhi_ref[i] = m[:, H:]


def _stage1_body(olo_ref, ohi_ref, rW_ref, rb_ref, w0_ref, w1_ref, w2_ref,
                 rlo_ref, rhi_ref, mlo_ref, mhi_ref):
    h = jnp.maximum(
        jnp.concatenate([olo_ref[...], ohi_ref[...]],
                        axis=1).astype(jnp.float32), 0.0)
    r = (jnp.dot(h, rW_ref[...], preferred_element_type=jnp.float32)
         + rb_ref[...]).astype(jnp.bfloat16)
    rlo_ref[...] = r[:, :H]
    rhi_ref[...] = r[:, H:]
    for i, w_ref in enumerate((w0_ref, w1_ref, w2_ref)):
        m = jnp.dot(h, w_ref[...],
                    preferred_element_type=jnp.float32).astype(jnp.bfloat16)
        mlo_ref[i] = m[:, :H]
        mhi_ref[i] = m[:, H:]


def _stage2_body(olo_ref, ohi_ref, w1_ref, b1_ref, w2_ref, b2_ref, o_ref):
    h = jnp.maximum(
        jnp.concatenate([olo_ref[...], ohi_ref[...]],
                        axis=1).astype(jnp.float32), 0.0)
    g = jnp.sum(h, axis=0, keepdims=True) * (1.0 / N)
    hh = jnp.maximum(
        jnp.dot(g, w1_ref[...], preferred_element_type=jnp.float32)
        + b1_ref[...], 0.0)
    o_ref[...] = (jnp.dot(hh, w2_ref[...], preferred_element_type=jnp.float32)
                  + b2_ref[...])


def _stage0(x, embW, embb, rW, rb, w0, w1, w2):
    nblk = N // BN
    return pl.pallas_call(
        _stage0_body,
        grid=(nblk,),
        in_specs=[
            pl.BlockSpec((BN, IN_FEAT), lambda b: (b, 0)),
            pl.BlockSpec((IN_FEAT, NHID), lambda b: (0, 0)),
            pl.BlockSpec((1, NHID), lambda b: (0, 0)),
            pl.BlockSpec((NHID, NHID), lambda b: (0, 0)),
            pl.BlockSpec((1, NHID), lambda b: (0, 0)),
            pl.BlockSpec((NHID, NHID), lambda b: (0, 0)),
            pl.BlockSpec((NHID, NHID), lambda b: (0, 0)),
            pl.BlockSpec((NHID, NHID), lambda b: (0, 0)),
        ],
        out_specs=[
            pl.BlockSpec((BN, H), lambda b: (b, 0)),
            pl.BlockSpec((BN, H), lambda b: (b, 0)),
            pl.BlockSpec((NREL, BN, H), lambda b: (0, b, 0)),
            pl.BlockSpec((NREL, BN, H), lambda b: (0, b, 0)),
        ],
        out_shape=[
            jax.ShapeDtypeStruct((N, H), jnp.bfloat16),
            jax.ShapeDtypeStruct((N, H), jnp.bfloat16),
            jax.ShapeDtypeStruct((NREL, N, H), jnp.bfloat16),
            jax.ShapeDtypeStruct((NREL, N, H), jnp.bfloat16),
        ],
    )(x, embW, embb, rW, rb, w0, w1, w2)


def _stage1(olo, ohi, rW, rb, w0, w1, w2):
    nblk = N // BN
    return pl.pallas_call(
        _stage1_body,
        grid=(nblk,),
        in_specs=[
            pl.BlockSpec((BN, H), lambda b: (b, 0)),
            pl.BlockSpec((BN, H), lambda b: (b, 0)),
            pl.BlockSpec((NHID, NHID), lambda b: (0, 0)),
            pl.BlockSpec((1, NHID), lambda b: (0, 0)),
            pl.BlockSpec((NHID, NHID), lambda b: (0, 0)),
            pl.BlockSpec((NHID, NHID), lambda b: (0, 0)),
            pl.BlockSpec((NHID, NHID), lambda b: (0, 0)),
        ],
        out_specs=[
            pl.BlockSpec((BN, H), lambda b: (b, 0)),
            pl.BlockSpec((BN, H), lambda b: (b, 0)),
            pl.BlockSpec((NREL, BN, H), lambda b: (0, b, 0)),
            pl.BlockSpec((NREL, BN, H), lambda b: (0, b, 0)),
        ],
        out_shape=[
            jax.ShapeDtypeStruct((N, H), jnp.bfloat16),
            jax.ShapeDtypeStruct((N, H), jnp.bfloat16),
            jax.ShapeDtypeStruct((NREL, N, H), jnp.bfloat16),
            jax.ShapeDtypeStruct((NREL, N, H), jnp.bfloat16),
        ],
    )(olo, ohi, rW, rb, w0, w1, w2)


def _stage2(olo, ohi, w1, b1, w2, b2):
    return pl.pallas_call(
        _stage2_body,
        out_shape=jax.ShapeDtypeStruct((1, 1), jnp.float32),
    )(olo, ohi, w1, b1, w2, b2)


def _sc_scatter(src0, dst0, src1, dst1, src2, dst2, r_lo, r_hi, m_lo, m_hi):
    mesh = plsc.VectorSubcoreMesh(core_axis_name="c", subcore_axis_name="s")

    @functools.partial(
        pl.kernel,
        mesh=mesh,
        out_type=[jax.ShapeDtypeStruct((N, H), jnp.bfloat16),
                  jax.ShapeDtypeStruct((N, H), jnp.bfloat16)],
        scratch_types=[
            pltpu.VMEM((2, G, CHUNK), jnp.int32),
            pltpu.VMEM((2, G, CHUNK), jnp.int32),
            pltpu.VMEM((2, G, CHUNK, H), jnp.bfloat16),
            pltpu.VMEM_SHARED((N, H), jnp.bfloat16),
            pltpu.SemaphoreType.DMA,
            pltpu.SemaphoreType.DMA,
            pltpu.SemaphoreType.DMA,
        ],
        compiler_params=pltpu.CompilerParams(use_tc_tiling_on_sc=False),
    )
    def scatter_kernel(s0, d0, s1, d1, s2, d2, rlo, rhi, mlo, mhi,
                       out_lo, out_hi, sv, dv, rows, acc, sem_i, sem_g, sem_s):
        c = lax.axis_index("c")
        s = lax.axis_index("s")
        node_lo = s * NODE_ROWS_PER_TILE

        def half(r_ref, m_ref, out_ref):
            pltpu.sync_copy(r_ref.at[pl.ds(node_lo, NODE_ROWS_PER_TILE)],
                            acc.at[pl.ds(node_lo, NODE_ROWS_PER_TILE)])
            plsc.subcore_barrier()
            row0 = s * ROWS_PER_TILE
            for i, (sr, dr) in enumerate(((s0, d0), (s1, d1), (s2, d2))):
                table = m_ref.at[i]

                def issue_idx(g, p, sr=sr, dr=dr):
                    base = row0 + g * G
                    pltpu.async_copy(sr.at[pl.ds(base, G)], sv.at[p], sem_i)
                    pltpu.async_copy(dr.at[pl.ds(base, G)], dv.at[p], sem_i)

                issue_idx(0, 0)

                def body(g, _, sr=sr, dr=dr, table=table, issue_idx=issue_idx):
                    p = lax.rem(g, 2)
                    base = row0 + g * G
                    pltpu.make_async_copy(sr.at[pl.ds(base, G)], sv.at[p],
                                          sem_i).wait()
                    pltpu.make_async_copy(dr.at[pl.ds(base, G)], dv.at[p],
                                          sem_i).wait()
                    gathers = [
                        pltpu.async_copy(table.at[sv.at[p, k]],
                                         rows.at[p, k], sem_g)
                        for k in range(G)]

                    @pl.when(g < NGRP - 1)
                    def _():
                        issue_idx(g + 1, 1 - p)

                    for h_ in gathers:
                        h_.wait()
                    scats = [
                        pltpu.async_copy(rows.at[p, k], acc.at[dv.at[p, k]],
                                         sem_s, add=True)
                        for k in range(G)]
                    for h_ in scats:
                        h_.wait()
                    return 0

                lax.fori_loop(0, NGRP, body, 0)

                @pl.when(s < ROWS_EXTRA)
                def _(sr=sr, dr=dr, table=table):
                    er = NTILES * ROWS_PER_TILE + s
                    pltpu.sync_copy(sr.at[er], sv.at[0, 0])
                    pltpu.sync_copy(dr.at[er], dv.at[0, 0])
                    pltpu.sync_copy(table.at[sv.at[0, 0]], rows.at[0, 0])
                    pltpu.sync_copy(rows.at[0, 0], acc.at[dv.at[0, 0]],
                                    add=True)
            plsc.subcore_barrier()
            pltpu.sync_copy(acc.at[pl.ds(node_lo, NODE_ROWS_PER_TILE)],
                            out_ref.at[pl.ds(node_lo, NODE_ROWS_PER_TILE)])

        @pl.when(c == 0)
        def _():
            half(rlo, mlo, out_lo)

        @pl.when(c == 1)
        def _():
            half(rhi, mhi, out_hi)

    return scatter_kernel(src0, dst0, src1, dst1, src2, dst2,
                          r_lo, r_hi, m_lo, m_hi)


def kernel(x, edge_index_0, edge_index_1, edge_index_2, batch, emb_W, emb_b,
           root_W_0, root_b_0, conv_W_0_0, conv_W_0_1, conv_W_0_2,
           root_W_1, root_b_1, conv_W_1_0, conv_W_1_1, conv_W_1_2,
           mlp_W1, mlp_b1, mlp_W2, mlp_b2):
    del batch
    edges = []
    for ei in (edge_index_0, edge_index_1, edge_index_2):
        edges.append(ei[0].reshape(ROWS, CHUNK))
        edges.append(ei[1].reshape(ROWS, CHUNK))

    r_lo, r_hi, m_lo, m_hi = _stage0(
        x, emb_W, emb_b.reshape(1, NHID), root_W_0, root_b_0.reshape(1, NHID),
        conv_W_0_0, conv_W_0_1, conv_W_0_2)
    out_lo, out_hi = _sc_scatter(*edges, r_lo, r_hi, m_lo, m_hi)

    r_lo, r_hi, m_lo, m_hi = _stage1(
        out_lo, out_hi, root_W_1, root_b_1.reshape(1, NHID),
        conv_W_1_0, conv_W_1_1, conv_W_1_2)
    out_lo, out_hi = _sc_scatter(*edges, r_lo, r_hi, m_lo, m_hi)

    o = _stage2(out_lo, out_hi, mlp_W1, mlp_b1.reshape(1, NHID),
                mlp_W2, mlp_b2.reshape(1, 1))
    return o.reshape(1)
